# speedup vs baseline: 2.9724x; 2.9724x over previous
import functools

import jax
import jax.numpy as jnp
from jax import lax
from jax.experimental import pallas as pl
from jax.experimental.pallas import tpu as pltpu
from jax.experimental.pallas import tpu_sc as plsc

EMBED_DIM = 128
NUM_CORES = 2
NUM_SUBCORES = 16
NUM_WORKERS = NUM_CORES * NUM_SUBCORES
CHUNK = 128


def _gather_body(n_chunks, idx_hbm, table_hbm, out_hbm, idx_v, rows_v, gsem):
    wid = lax.axis_index("s") * NUM_CORES + lax.axis_index("c")
    pltpu.sync_copy(idx_hbm.at[wid], idx_v)

    def chunk(j, carry):
        pltpu.async_copy(table_hbm.at[idx_v.at[j]], rows_v, gsem).wait()
        pltpu.sync_copy(
            rows_v, out_hbm.at[pl.ds((wid * n_chunks + j) * CHUNK, CHUNK)]
        )
        return carry

    lax.fori_loop(0, n_chunks, chunk, 0)


def kernel(x, table):
    B, S = x.shape
    total = B * S
    assert total % (NUM_WORKERS * CHUNK) == 0
    n_chunks = total // (NUM_WORKERS * CHUNK)
    idx = x.reshape(NUM_WORKERS, n_chunks, CHUNK).astype(jnp.int32)

    mesh = plsc.VectorSubcoreMesh(core_axis_name="c", subcore_axis_name="s")
    k = pl.kernel(
        functools.partial(_gather_body, n_chunks),
        out_type=jax.ShapeDtypeStruct((total, EMBED_DIM), jnp.float32),
        mesh=mesh,
        scratch_types=[
            pltpu.VMEM((n_chunks, CHUNK), jnp.int32),
            pltpu.VMEM((CHUNK, EMBED_DIM), jnp.float32),
            pltpu.SemaphoreType.DMA,
        ],
    )
    out = k(idx, table)
    return out.reshape(B, S, EMBED_DIM)


# speedup vs baseline: 3.3144x; 1.1151x over previous
import functools

import jax
import jax.numpy as jnp
from jax import lax
from jax.experimental import pallas as pl
from jax.experimental.pallas import tpu as pltpu
from jax.experimental.pallas import tpu_sc as plsc

EMBED_DIM = 128
NUM_CORES = 2
NUM_SUBCORES = 16
NUM_WORKERS = NUM_CORES * NUM_SUBCORES
CHUNK = 128
K = 5


def _gather_body(n_chunks, idx_hbm, table_hbm, out_hbm, idx_v, rows_v, *sems):
    gsems, ssems = sems[:K], sems[K:]
    wid = lax.axis_index("s") * NUM_CORES + lax.axis_index("c")
    base = wid * n_chunks
    pltpu.sync_copy(idx_hbm.at[wid], idx_v)

    def gstart(j, b):
        pltpu.async_copy(table_hbm.at[idx_v.at[j]], rows_v.at[b], gsems[b])

    def gwait(b):
        pltpu.make_async_copy(
            table_hbm.at[idx_v.at[0]], rows_v.at[b], gsems[b]
        ).wait()

    def sstart(j, b):
        pltpu.async_copy(
            rows_v.at[b], out_hbm.at[pl.ds((base + j) * CHUNK, CHUNK)], ssems[b]
        )

    def swait(b):
        pltpu.make_async_copy(
            rows_v.at[b], out_hbm.at[pl.ds(0, CHUNK)], ssems[b]
        ).wait()

    for b in range(K):
        gstart(b, b)

    n_outer = n_chunks // K

    def outer(i, carry):
        j0 = i * K
        for b in range(K):
            gwait(b)
            sstart(j0 + b, b)
        for b in range(K):
            swait(b)
            gstart(j0 + K + b, b)
        return carry

    lax.fori_loop(0, n_outer - 1, outer, 0)

    j0 = (n_outer - 1) * K
    for b in range(K):
        gwait(b)
        sstart(j0 + b, b)
    for b in range(K):
        swait(b)


def kernel(x, table):
    B, S = x.shape
    total = B * S
    assert total % (NUM_WORKERS * CHUNK) == 0
    n_chunks = total // (NUM_WORKERS * CHUNK)
    assert n_chunks % K == 0
    idx = x.reshape(NUM_WORKERS, n_chunks, CHUNK).astype(jnp.int32)

    mesh = plsc.VectorSubcoreMesh(core_axis_name="c", subcore_axis_name="s")
    k = pl.kernel(
        functools.partial(_gather_body, n_chunks),
        out_type=jax.ShapeDtypeStruct((total, EMBED_DIM), jnp.float32),
        mesh=mesh,
        scratch_types=[
            pltpu.VMEM((n_chunks, CHUNK), jnp.int32),
            pltpu.VMEM((K, CHUNK, EMBED_DIM), jnp.float32),
        ]
        + [pltpu.SemaphoreType.DMA] * (2 * K),
    )
    out = k(idx, table)
    return out.reshape(B, S, EMBED_DIM)


# speedup vs baseline: 5.9402x; 1.7922x over previous
import functools

import jax
import jax.numpy as jnp
from jax import lax
from jax.experimental import pallas as pl
from jax.experimental.pallas import tpu as pltpu
from jax.experimental.pallas import tpu_sc as plsc

EMBED_DIM = 128
NUM_CORES = 2
NUM_SUBCORES = 16
NUM_WORKERS = NUM_CORES * NUM_SUBCORES
K = 8


def _gather_body(b_per_w, seq, idx_hbm, table_hbm, out_hbm, idx_v, rows_v, *sems):
    gsems, ssems = sems[:K], sems[K:]
    wid = lax.axis_index("s") * NUM_CORES + lax.axis_index("c")
    base = wid * b_per_w
    pltpu.sync_copy(idx_hbm.at[pl.ds(base, b_per_w)], idx_v)

    def gstart(j, b):
        pltpu.async_copy(table_hbm.at[idx_v.at[j]], rows_v.at[b], gsems[b])

    def gwait(b):
        pltpu.make_async_copy(
            table_hbm.at[idx_v.at[0]], rows_v.at[b], gsems[b]
        ).wait()

    def sstart(j, b):
        pltpu.async_copy(rows_v.at[b], out_hbm.at[base + j], ssems[b])

    def swait(b):
        pltpu.make_async_copy(rows_v.at[b], out_hbm.at[0], ssems[b]).wait()

    for b in range(K):
        gstart(b, b)

    n_outer = b_per_w // K

    def outer(i, carry):
        j0 = i * K
        for b in range(K):
            gwait(b)
            sstart(j0 + b, b)
        for b in range(K):
            swait(b)
            gstart(j0 + K + b, b)
        return carry

    lax.fori_loop(0, n_outer - 1, outer, 0)

    j0 = (n_outer - 1) * K
    for b in range(K):
        gwait(b)
        sstart(j0 + b, b)
    for b in range(K):
        swait(b)


def kernel(x, table):
    B, S = x.shape
    assert B % NUM_WORKERS == 0
    b_per_w = B // NUM_WORKERS
    assert b_per_w % K == 0
    idx = x.astype(jnp.int32)

    mesh = plsc.VectorSubcoreMesh(core_axis_name="c", subcore_axis_name="s")
    k = pl.kernel(
        functools.partial(_gather_body, b_per_w, S),
        out_type=jax.ShapeDtypeStruct((B, S, EMBED_DIM), jnp.float32),
        mesh=mesh,
        scratch_types=[
            pltpu.VMEM((b_per_w, S), jnp.int32),
            pltpu.VMEM((K, S, EMBED_DIM), jnp.float32),
        ]
        + [pltpu.SemaphoreType.DMA] * (2 * K),
    )
    return k(idx, table)
